# baseline (speedup 1.0000x reference)
import functools
import math

import jax
import jax.numpy as jnp
import numpy as np
from jax import lax
from jax.experimental import pallas as pl
from jax.experimental.pallas import tpu as pltpu
from jax.experimental.pallas import tpu_sc as plsc

L_AG = 5
C = 128
H_FEAT = 128
W_FEAT = 256
HW = H_FEAT * W_FEAT

NC = 2
NS = 16
NW = NC * NS
H_PER_W = H_FEAT // NW
P = 128
WBLK = W_FEAT // P
NCHUNK = H_PER_W * WBLK
NG = P // 16

_SQRT_C = float(np.float32(np.sqrt(np.float32(C))))
_COEF = {
    s: math.factorial(s - 1) * math.factorial(L_AG - s) / math.factorial(L_AG)
    for s in range(2, L_AG + 1)
}
_SINGLE = 1.0 / L_AG - sum(
    math.factorial(r) * math.factorial(L_AG - r - 1) / math.factorial(L_AG)
    * math.comb(L_AG - 1, r)
    for r in range(1, L_AG)
)

_NPARAM = 2 * C + 2
_PLEN = ((_NPARAM * 16 + 127) // 128) * 128

_LOG2E = 1.4426950408889634
_LN2 = 0.6931471805599453
_MAGIC = 12582912.0


def _rb(x):
    cc = x * 65537.0
    return cc - (cc - x)


def _pexp(x):
    z = jnp.maximum(x * _LOG2E, -30.0)
    r = (z + _MAGIC) - _MAGIC
    t = (z - r) * _LN2
    p = jnp.full((16,), 1.0 / 5040.0, jnp.float32)
    for coef in (1.0 / 720, 1.0 / 120, 1.0 / 24, 1.0 / 6, 0.5, 1.0, 1.0):
        p = p * t + coef
    j = (-r).astype(jnp.int32)
    sf = (1 << j).astype(jnp.float32)
    return p / sf


def _fusion_body(feats_hbm, params_hbm, out_hbm, wbuf, rbuf, obuf, pbuf,
                 osem):
    wid = lax.axis_index("s") * NC + lax.axis_index("c")
    pltpu.sync_copy(params_hbm, pbuf)
    mlp_b = pbuf[pl.ds(16 * (2 * C), 16)]
    cb = pbuf[pl.ds(16 * (2 * C + 1), 16)]

    def chunk(k, carry):
        h = wid * H_PER_W + k // WBLK
        wb = (k % WBLK) * P
        slot = lax.rem(k, 2)
        pltpu.sync_copy(feats_hbm.at[:, :, h, pl.ds(wb, P)], wbuf)

        @pl.when(k >= 2)
        def _drain():
            pltpu.make_async_copy(
                out_hbm.at[:, h, pl.ds(wb, P)], obuf.at[slot], osem).wait()
        for g in range(NG):
            g16 = g * 16
            z = jnp.zeros((16,), jnp.float32)

            def abody(ci, acc):
                d1, d2, d3, d4, e0, e1, e2, e3, e4 = acc
                for u in range(2):
                    c = ci * 2 + u
                    b0 = _rb(wbuf[0, c, pl.ds(g16, 16)])
                    b1 = _rb(wbuf[1, c, pl.ds(g16, 16)])
                    b2 = _rb(wbuf[2, c, pl.ds(g16, 16)])
                    b3 = _rb(wbuf[3, c, pl.ds(g16, 16)])
                    b4 = _rb(wbuf[4, c, pl.ds(g16, 16)])
                    rbuf[pl.ds(c * 16, 16)] = b1
                    rbuf[pl.ds((C + c) * 16, 16)] = b2
                    rbuf[pl.ds((2 * C + c) * 16, 16)] = b3
                    rbuf[pl.ds((3 * C + c) * 16, 16)] = b4
                    cc = pbuf[pl.ds((C + c) * 16, 16)]
                    d1 = d1 + b0 * b1
                    d2 = d2 + b0 * b2
                    d3 = d3 + b0 * b3
                    d4 = d4 + b0 * b4
                    e0 = e0 + b0 * cc
                    e1 = e1 + b1 * cc
                    e2 = e2 + b2 * cc
                    e3 = e3 + b3 * cc
                    e4 = e4 + b4 * cc
                return (d1, d2, d3, d4, e0, e1, e2, e3, e4)

            (d1, d2, d3, d4, e0, e1, e2, e3, e4) = lax.fori_loop(
                0, C // 2, abody, (z, z, z, z, z, z, z, z, z))

            d1 = d1 / _SQRT_C
            d2 = d2 / _SQRT_C
            d3 = d3 / _SQRT_C
            d4 = d4 / _SQRT_C
            mx = jnp.maximum(jnp.maximum(d1, d2), jnp.maximum(d3, d4))
            x1 = _pexp(d1 - mx)
            x2 = _pexp(d2 - mx)
            x3 = _pexp(d3 - mx)
            x4 = _pexp(d4 - mx)
            xs = x1 + x2 + x3 + x4
            a1 = _rb(x1 / xs)
            a2 = _rb(x2 / xs)
            a3 = _rb(x3 / xs)
            a4 = _rb(x4 / xs)

            def tbody(ci, tacc):
                for u in range(2):
                    c = ci * 2 + u
                    b1 = rbuf[pl.ds(c * 16, 16)]
                    b2 = rbuf[pl.ds((C + c) * 16, 16)]
                    b3 = rbuf[pl.ds((2 * C + c) * 16, 16)]
                    b4 = rbuf[pl.ds((3 * C + c) * 16, 16)]
                    ctx = a1 * b1 + a2 * b2 + a3 * b3 + a4 * b4
                    wc = pbuf[pl.ds(c * 16, 16)]
                    tacc = tacc + _rb(ctx) * wc
                return tacc

            t = lax.fori_loop(0, C // 2, tbody, z) + mlp_b
            mask = jnp.where(t > 0.0, 1.0, 0.0).astype(jnp.float32)

            gs = (e0, mask * e1, mask * e2, mask * e3, mask * e4)

            ssum = {}
            r = {}
            for T in range(1, 32):
                lb = T & (-T)
                rest = T ^ lb
                s_T = gs[lb.bit_length() - 1]
                if rest:
                    s_T = s_T + ssum[rest]
                ssum[T] = s_T
                r[T] = jnp.maximum(s_T + cb, 0.0)

            sv = []
            for i in range(L_AG):
                bit = 1 << i
                acc = _SINGLE * r[bit]
                for T in range(1, 32):
                    if (T & bit) and T != bit:
                        acc = acc + _COEF[bin(T).count("1")] * r[T]
                sv.append(acc)

            m2 = sv[0]
            for i in range(1, L_AG):
                m2 = jnp.maximum(m2, sv[i])
            p = [_pexp(v - m2) for v in sv]
            tot = p[0] + p[1] + p[2] + p[3] + p[4]
            inv = 1.0 / tot
            minv = mask * inv
            q0 = p[0] * inv
            q1 = p[1] * minv
            q2 = p[2] * minv
            q3 = p[3] * minv
            q4 = p[4] * minv

            def obody(ci, carry2):
                for u in range(2):
                    c = ci * 2 + u
                    v0 = wbuf[0, c, pl.ds(g16, 16)]
                    v1 = wbuf[1, c, pl.ds(g16, 16)]
                    v2 = wbuf[2, c, pl.ds(g16, 16)]
                    v3 = wbuf[3, c, pl.ds(g16, 16)]
                    v4 = wbuf[4, c, pl.ds(g16, 16)]
                    obuf[slot, c, pl.ds(g16, 16)] = (
                        q0 * v0 + q1 * v1 + q2 * v2 + q3 * v3 + q4 * v4)
                return carry2

            lax.fori_loop(0, C // 2, obody, 0)

        pltpu.async_copy(obuf.at[slot], out_hbm.at[:, h, pl.ds(wb, P)], osem)
        return carry

    lax.fori_loop(0, NCHUNK, chunk, 0)
    for i in range(2):
        pltpu.make_async_copy(
            out_hbm.at[:, 0, pl.ds(0, P)], obuf.at[i], osem).wait()


_sc_fusion = functools.partial(
    pl.kernel,
    out_type=jax.ShapeDtypeStruct((C, H_FEAT, W_FEAT), jnp.float32),
    mesh=plsc.VectorSubcoreMesh(core_axis_name="c", subcore_axis_name="s"),
    scratch_types=[
        pltpu.VMEM((L_AG, C, P), jnp.float32),
        pltpu.VMEM(((L_AG - 1) * C * 16,), jnp.float32),
        pltpu.VMEM((2, C, P), jnp.float32),
        pltpu.VMEM((_PLEN,), jnp.float32),
        pltpu.SemaphoreType.DMA,
    ],
)(_fusion_body)


def _round_bf16_host(x):
    cc = x * jnp.float32(65537.0)
    return cc - (cc - x)


def kernel(feats, mlp_w, mlp_b, contrib_w, contrib_b):
    pvec = jnp.concatenate([
        _round_bf16_host(mlp_w.reshape(-1)),
        _round_bf16_host(contrib_w.reshape(-1)),
        mlp_b.reshape(-1), contrib_b.reshape(-1),
    ])
    params = jnp.broadcast_to(pvec[:, None], (_NPARAM, 16)).reshape(-1)
    params = jnp.concatenate(
        [params, jnp.zeros((_PLEN - _NPARAM * 16,), jnp.float32)])
    return _sc_fusion(feats, params)


# speedup vs baseline: 1.2599x; 1.2599x over previous
import functools
import math

import jax
import jax.numpy as jnp
import numpy as np
from jax import lax
from jax.experimental import pallas as pl
from jax.experimental.pallas import tpu as pltpu
from jax.experimental.pallas import tpu_sc as plsc

L_AG = 5
C = 128
H_FEAT = 128
W_FEAT = 256
HW = H_FEAT * W_FEAT

NC = 2
NS = 16
NW = NC * NS
H_PER_W = H_FEAT // NW
P = 128
WBLK = W_FEAT // P
NCHUNK = H_PER_W * WBLK
NG = P // 16

_SQRT_C = float(np.float32(np.sqrt(np.float32(C))))
_COEF = {
    s: math.factorial(s - 1) * math.factorial(L_AG - s) / math.factorial(L_AG)
    for s in range(2, L_AG + 1)
}
_SINGLE = 1.0 / L_AG - sum(
    math.factorial(r) * math.factorial(L_AG - r - 1) / math.factorial(L_AG)
    * math.comb(L_AG - 1, r)
    for r in range(1, L_AG)
)

_NPARAM = 2 * C + 2
_PLEN = ((_NPARAM * 16 + 127) // 128) * 128

_LOG2E = 1.4426950408889634
_LN2 = 0.6931471805599453
_MAGIC = 12582912.0


def _rb(x):
    cc = x * 65537.0
    return cc - (cc - x)


def _pexp(x):
    z = jnp.maximum(x * _LOG2E, -30.0)
    r = (z + _MAGIC) - _MAGIC
    t = (z - r) * _LN2
    p = jnp.full((16,), 1.0 / 5040.0, jnp.float32)
    for coef in (1.0 / 720, 1.0 / 120, 1.0 / 24, 1.0 / 6, 0.5, 1.0, 1.0):
        p = p * t + coef
    j = (-r).astype(jnp.int32)
    sf = (1 << j).astype(jnp.float32)
    return p / sf


def _fusion_body(feats_hbm, params_hbm, out_hbm, wbuf, rbuf, obuf, pbuf):
    wid = lax.axis_index("s") * NC + lax.axis_index("c")
    pltpu.sync_copy(params_hbm, pbuf)
    mlp_b = pbuf[pl.ds(16 * (2 * C), 16)]
    cb = pbuf[pl.ds(16 * (2 * C + 1), 16)]

    def chunk(k, carry):
        h = wid * H_PER_W + k // WBLK
        wb = (k % WBLK) * P
        pltpu.sync_copy(feats_hbm.at[:, :, h, pl.ds(wb, P)], wbuf)
        for g in range(NG):
            g16 = g * 16
            z = jnp.zeros((16,), jnp.float32)

            def abody(ci, acc):
                d1, d2, d3, d4, e0, e1, e2, e3, e4 = acc
                for u in range(2):
                    c = ci * 2 + u
                    b0 = _rb(wbuf[0, c, pl.ds(g16, 16)])
                    b1 = _rb(wbuf[1, c, pl.ds(g16, 16)])
                    b2 = _rb(wbuf[2, c, pl.ds(g16, 16)])
                    b3 = _rb(wbuf[3, c, pl.ds(g16, 16)])
                    b4 = _rb(wbuf[4, c, pl.ds(g16, 16)])
                    rbuf[pl.ds(c * 16, 16)] = b1
                    rbuf[pl.ds((C + c) * 16, 16)] = b2
                    rbuf[pl.ds((2 * C + c) * 16, 16)] = b3
                    rbuf[pl.ds((3 * C + c) * 16, 16)] = b4
                    cc = pbuf[pl.ds((C + c) * 16, 16)]
                    d1 = d1 + b0 * b1
                    d2 = d2 + b0 * b2
                    d3 = d3 + b0 * b3
                    d4 = d4 + b0 * b4
                    e0 = e0 + b0 * cc
                    e1 = e1 + b1 * cc
                    e2 = e2 + b2 * cc
                    e3 = e3 + b3 * cc
                    e4 = e4 + b4 * cc
                return (d1, d2, d3, d4, e0, e1, e2, e3, e4)

            (d1, d2, d3, d4, e0, e1, e2, e3, e4) = lax.fori_loop(
                0, C // 2, abody, (z, z, z, z, z, z, z, z, z))

            d1 = d1 / _SQRT_C
            d2 = d2 / _SQRT_C
            d3 = d3 / _SQRT_C
            d4 = d4 / _SQRT_C
            mx = jnp.maximum(jnp.maximum(d1, d2), jnp.maximum(d3, d4))
            x1 = _pexp(d1 - mx)
            x2 = _pexp(d2 - mx)
            x3 = _pexp(d3 - mx)
            x4 = _pexp(d4 - mx)
            xs = x1 + x2 + x3 + x4
            a1 = _rb(x1 / xs)
            a2 = _rb(x2 / xs)
            a3 = _rb(x3 / xs)
            a4 = _rb(x4 / xs)

            def tbody(ci, tacc):
                for u in range(2):
                    c = ci * 2 + u
                    b1 = rbuf[pl.ds(c * 16, 16)]
                    b2 = rbuf[pl.ds((C + c) * 16, 16)]
                    b3 = rbuf[pl.ds((2 * C + c) * 16, 16)]
                    b4 = rbuf[pl.ds((3 * C + c) * 16, 16)]
                    ctx = a1 * b1 + a2 * b2 + a3 * b3 + a4 * b4
                    wc = pbuf[pl.ds(c * 16, 16)]
                    tacc = tacc + _rb(ctx) * wc
                return tacc

            t = lax.fori_loop(0, C // 2, tbody, z) + mlp_b
            mask = jnp.where(t > 0.0, 1.0, 0.0).astype(jnp.float32)

            gs = (e0, mask * e1, mask * e2, mask * e3, mask * e4)

            ssum = {}
            r = {}
            for T in range(1, 32):
                lb = T & (-T)
                rest = T ^ lb
                s_T = gs[lb.bit_length() - 1]
                if rest:
                    s_T = s_T + ssum[rest]
                ssum[T] = s_T
                r[T] = jnp.maximum(s_T + cb, 0.0)

            sv = []
            for i in range(L_AG):
                bit = 1 << i
                acc = _SINGLE * r[bit]
                for T in range(1, 32):
                    if (T & bit) and T != bit:
                        acc = acc + _COEF[bin(T).count("1")] * r[T]
                sv.append(acc)

            m2 = sv[0]
            for i in range(1, L_AG):
                m2 = jnp.maximum(m2, sv[i])
            p = [_pexp(v - m2) for v in sv]
            tot = p[0] + p[1] + p[2] + p[3] + p[4]
            inv = 1.0 / tot
            minv = mask * inv
            q0 = p[0] * inv
            q1 = p[1] * minv
            q2 = p[2] * minv
            q3 = p[3] * minv
            q4 = p[4] * minv

            def obody(ci, carry2):
                for u in range(2):
                    c = ci * 2 + u
                    v0 = wbuf[0, c, pl.ds(g16, 16)]
                    v1 = wbuf[1, c, pl.ds(g16, 16)]
                    v2 = wbuf[2, c, pl.ds(g16, 16)]
                    v3 = wbuf[3, c, pl.ds(g16, 16)]
                    v4 = wbuf[4, c, pl.ds(g16, 16)]
                    obuf[c, pl.ds(g16, 16)] = (
                        q0 * v0 + q1 * v1 + q2 * v2 + q3 * v3 + q4 * v4)
                return carry2

            lax.fori_loop(0, C // 2, obody, 0)

        pltpu.sync_copy(obuf, out_hbm.at[:, h, pl.ds(wb, P)])
        return carry

    lax.fori_loop(0, NCHUNK, chunk, 0)


_sc_fusion = functools.partial(
    pl.kernel,
    out_type=jax.ShapeDtypeStruct((C, H_FEAT, W_FEAT), jnp.float32),
    mesh=plsc.VectorSubcoreMesh(core_axis_name="c", subcore_axis_name="s"),
    scratch_types=[
        pltpu.VMEM((L_AG, C, P), jnp.float32),
        pltpu.VMEM(((L_AG - 1) * C * 16,), jnp.float32),
        pltpu.VMEM((C, P), jnp.float32),
        pltpu.VMEM((_PLEN,), jnp.float32),
    ],
)(_fusion_body)


def _round_bf16_host(x):
    cc = x * jnp.float32(65537.0)
    return cc - (cc - x)


def kernel(feats, mlp_w, mlp_b, contrib_w, contrib_b):
    pvec = jnp.concatenate([
        _round_bf16_host(mlp_w.reshape(-1)),
        _round_bf16_host(contrib_w.reshape(-1)),
        mlp_b.reshape(-1), contrib_b.reshape(-1),
    ])
    params = jnp.broadcast_to(pvec[:, None], (_NPARAM, 16)).reshape(-1)
    params = jnp.concatenate(
        [params, jnp.zeros((_PLEN - _NPARAM * 16,), jnp.float32)])
    return _sc_fusion(feats, params)
